# baseline (speedup 1.0000x reference)
import jax
import jax.numpy as jnp
from jax import lax
from jax.experimental import pallas as pl
from jax.experimental.pallas import tpu as pltpu
from jax.experimental.pallas import tpu_sc as plsc

R = 262144
K = 512
C = 3
NC = 2
NS = 16
L = 16
NW = NC * NS
ROWS_PER_W = R // NW
CHUNK = 64
NCHUNK = ROWS_PER_W // CHUNK
GROUPS = CHUNK // L
BLOCKS = K // L


def _sc_body(w_hbm, pal_hbm, out_hbm, buf0, buf1, pal_v, out_v, sem0, sem1):
    cid = lax.axis_index("c")
    sid = lax.axis_index("s")
    wid = sid * NC + cid
    row0 = wid * ROWS_PER_W

    pltpu.sync_copy(pal_hbm, pal_v)

    def in_slice(t):
        return w_hbm.at[pl.ds((row0 + t * CHUNK) * K, CHUNK * K)]

    pltpu.async_copy(in_slice(0), buf0, sem0)
    pltpu.async_copy(in_slice(1), buf1, sem1)

    iota = lax.iota(jnp.int32, L)

    def process_chunk(t, buf):
        for g in range(GROUPS):
            base = (g * L + iota) * K
            m0 = jnp.full((L,), -jnp.inf, jnp.float32)
            b0 = jnp.zeros((L,), jnp.int32)

            def blk(_, carry):
                m, bidx, idx0 = carry
                for u in range(L):
                    idxu = idx0 + u
                    v = plsc.load_gather(buf, [idxu])
                    upd = v > m
                    m = jnp.maximum(m, v)
                    bidx = jnp.where(upd, idxu, bidx)
                return m, bidx, idx0 + L

            _, bidx, _ = lax.fori_loop(0, BLOCKS, blk, (m0, b0, base))
            col3 = jnp.bitwise_and(bidx, K - 1) * C
            ob = (t * CHUNK + g * L + iota) * C
            for c in range(C):
                ch = plsc.load_gather(pal_v, [col3 + c])
                plsc.store_scatter(out_v, [ob + c], ch)

    def pair(t2, carry):
        for b, (buf, sem) in enumerate(((buf0, sem0), (buf1, sem1))):
            t = t2 * 2 + b
            pltpu.make_async_copy(in_slice(t), buf, sem).wait()
            process_chunk(t, buf)

            @pl.when(t + 2 < NCHUNK)
            def _():
                pltpu.async_copy(in_slice(t + 2), buf, sem)

        return carry

    lax.fori_loop(0, NCHUNK // 2, pair, 0)

    pltpu.sync_copy(out_v, out_hbm.at[pl.ds(row0 * C, ROWS_PER_W * C)])


_sc_call = pl.kernel(
    _sc_body,
    out_type=jax.ShapeDtypeStruct((R * C,), jnp.float32),
    mesh=plsc.VectorSubcoreMesh(core_axis_name="c", subcore_axis_name="s"),
    compiler_params=pltpu.CompilerParams(needs_layout_passes=False),
    scratch_types=[
        pltpu.VMEM((CHUNK * K,), jnp.float32),
        pltpu.VMEM((CHUNK * K,), jnp.float32),
        pltpu.VMEM((K * C,), jnp.float32),
        pltpu.VMEM((ROWS_PER_W * C,), jnp.float32),
        pltpu.SemaphoreType.DMA,
        pltpu.SemaphoreType.DMA,
    ],
)


def kernel(raw_weights, palette):
    out = _sc_call(raw_weights.reshape(-1), palette.reshape(-1))
    return out.reshape(R, C)
